# baseline (speedup 1.0000x reference)
import jax
import jax.numpy as jnp
from jax import lax
from jax.experimental import pallas as pl
from jax.experimental.pallas import tpu as pltpu
from jax.experimental.pallas import tpu_sc as plsc

NC = 2
NS = 16
L = 16
CW = 8

CH = 40
NBUF = 8
PF = 4
SD = 6

_ZR = 1000


def _sc_aggregate(n_nodes, d, n_edges, with_counts):
    nw = NC * NS
    ept = n_edges // nw
    assert ept * nw == n_edges and ept % CH == 0
    n_chunks = ept // CH
    assert PF < SD < NBUF
    assert n_nodes % _ZR == 0

    mesh = plsc.VectorSubcoreMesh(core_axis_name="c", subcore_axis_name="s")

    out_type = [jax.ShapeDtypeStruct((NC, n_nodes, d), jnp.float32)]
    if with_counts:
        out_type.append(jax.ShapeDtypeStruct((NC, n_nodes), jnp.float32))

    scratch = [
        pltpu.VMEM_SHARED((n_nodes, d), jnp.float32),
    ]
    scratch += [pltpu.SemaphoreType.DMA for _ in range(3 * NBUF)]
    scratch += [pltpu.VMEM((CH, d), jnp.float32) for _ in range(NBUF)]
    scratch += [pltpu.VMEM((CH,), jnp.int32) for _ in range(NBUF)]
    scratch += [pltpu.VMEM((CH,), jnp.int32) for _ in range(NBUF)]
    if with_counts:
        scratch += [
            pltpu.VMEM_SHARED((n_nodes,), jnp.float32),
            pltpu.VMEM(((CH + L - 1) // L * L,), jnp.float32),
        ]

    def body(feat, src, dst, zf, z1, *rest):
        k = 1 + (1 if with_counts else 0)
        parts = rest[0]
        cnts = rest[1] if with_counts else None
        agg_sh = rest[k]
        p = k + 1
        gsem = list(rest[p:p + NBUF])
        dsem = list(rest[p + NBUF:p + 2 * NBUF])
        ssem = list(rest[p + 2 * NBUF:p + 3 * NBUF])
        p += 3 * NBUF
        rows = list(rest[p:p + NBUF])
        dring = list(rest[p + NBUF:p + 2 * NBUF])
        sring = list(rest[p + 2 * NBUF:p + 3 * NBUF])
        if with_counts:
            cnt_sh, ones_v = rest[p + 3 * NBUF:]
        else:
            cnt_sh = ones_v = None
        cid = lax.axis_index("c")
        sid = lax.axis_index("s")
        wid = cid * NS + sid
        base = wid * ept

        def issue_idx(c, slot):
            pltpu.async_copy(src.at[pl.ds(base + c * CH, CH)], sring[slot],
                             ssem[slot])
            pltpu.async_copy(dst.at[pl.ds(base + c * CH, CH)], dring[slot],
                             dsem[slot])

        def issue_gather(c, slot):
            pltpu.make_async_copy(src.at[pl.ds(base + c * CH, CH)],
                                  sring[slot], ssem[slot]).wait()
            pltpu.async_copy(feat.at[sring[slot]], rows[slot],
                             gsem[slot])

        for c0 in range(SD):
            issue_idx(c0, c0)
        for c0 in range(PF):
            issue_gather(c0, c0)

        r_lo = n_nodes // NS // 8 * 8
        r_hi = n_nodes - r_lo * (NS - 1)
        row0 = sid * r_lo

        @pl.when(sid < NS - 1)
        def _():
            pltpu.sync_copy(zf.at[pl.ds(0, r_lo)],
                            agg_sh.at[pl.ds(row0, r_lo)])

        @pl.when(sid == NS - 1)
        def _():
            pltpu.sync_copy(zf.at[pl.ds(0, r_hi)],
                            agg_sh.at[pl.ds(row0, r_hi)])
        if with_counts:
            @pl.when(sid == 0)
            def _():
                pltpu.sync_copy(z1, cnt_sh)
        if with_counts:
            for q in range((CH + L - 1) // L):
                ones_v[pl.ds(q * L, L)] = jnp.ones((L,), jnp.float32)
        plsc.subcore_barrier()

        def turn(c, j):
            ij = (j + SD) % NBUF
            gj = (j + PF) % NBUF

            @pl.when(c + SD < n_chunks)
            def _():
                issue_idx(c + SD, ij)

            @pl.when(c + PF < n_chunks)
            def _():
                issue_gather(c + PF, gj)
            pltpu.make_async_copy(feat.at[sring[j]], rows[j],
                                  gsem[j]).wait()
            pltpu.make_async_copy(dst.at[pl.ds(base + c * CH, CH)], dring[j],
                                  dsem[j]).wait()
            pltpu.sync_copy(rows[j], agg_sh.at[dring[j]], add=True)
            if with_counts:
                pltpu.sync_copy(ones_v.at[pl.ds(0, CH)],
                                cnt_sh.at[dring[j]], add=True)

        def super_turn(t, _):
            for j in range(NBUF):
                turn(t * NBUF + j, j)
            return 0

        n_full = n_chunks // NBUF
        lax.fori_loop(0, n_full, super_turn, 0)
        for jj in range(n_chunks % NBUF):
            turn(n_full * NBUF + jj, jj)

        plsc.subcore_barrier()

        @pl.when(sid < NS - 1)
        def _():
            pltpu.sync_copy(agg_sh.at[pl.ds(row0, r_lo)],
                            parts.at[cid, pl.ds(row0, r_lo)])

        @pl.when(sid == NS - 1)
        def _():
            pltpu.sync_copy(agg_sh.at[pl.ds(row0, r_hi)],
                            parts.at[cid, pl.ds(row0, r_hi)])
        if with_counts:
            @pl.when(sid == 0)
            def _():
                pltpu.sync_copy(cnt_sh, cnts.at[cid])

    kern = pl.kernel(body, out_type=tuple(out_type), mesh=mesh,
                     scratch_types=scratch)
    return kern


_BLK = 1000


def _tc_split(ei, d):
    e = ei.shape[1]

    def body(ei_r, s_r, d_r, zf_r):
        s_r[...] = ei_r[0]
        d_r[...] = ei_r[1]
        zf_r[...] = jnp.zeros_like(zf_r)

    return pl.pallas_call(
        body,
        out_shape=[
            jax.ShapeDtypeStruct((e,), jnp.int32),
            jax.ShapeDtypeStruct((e,), jnp.int32),
            jax.ShapeDtypeStruct((_ZR, d), jnp.float32),
        ],
    )(ei)


def _tc_self(x_self, W_r, b):
    n, d = x_self.shape
    h = W_r.shape[1]

    def body(x_r, wr_r, b_r, o_r):
        o_r[...] = (jnp.dot(x_r[...], wr_r[...],
                            preferred_element_type=jnp.float32) + b_r[...])

    grid = (n // _BLK,)
    row_spec = pl.BlockSpec((_BLK, d), lambda i: (i, 0))
    w_spec = pl.BlockSpec((d, h), lambda i: (0, 0))
    b_spec = pl.BlockSpec((1, h), lambda i: (0, 0))
    return pl.pallas_call(
        body,
        grid=grid,
        in_specs=[row_spec, w_spec, b_spec],
        out_specs=pl.BlockSpec((_BLK, h), lambda i: (i, 0)),
        out_shape=jax.ShapeDtypeStruct((n, h), jnp.float32),
    )(x_self, W_r, b.reshape(1, h))


def _tc_combine(parts, cnts, t_self, W_l, relu):
    n, h = t_self.shape
    d = parts.shape[2]
    blk = _BLK
    assert n % blk == 0

    def body(p_r, c_r, t_r, wl_r, o_r):
        agg = p_r[0] + p_r[1]
        cnt = c_r[0] + c_r[1]
        recip = (1.0 / jnp.maximum(cnt, 1.0))[:, None]
        mean = agg * recip
        out = (jnp.dot(mean, wl_r[...], preferred_element_type=jnp.float32)
               + t_r[...])
        if relu:
            out = jnp.maximum(out, 0.0)
        o_r[...] = out

    return pl.pallas_call(
        body,
        out_shape=jax.ShapeDtypeStruct((n, h), jnp.float32),
    )(parts, cnts, t_self, W_l)


def kernel(x, edge_index, W1_l, b1, W1_r, W2_l, b2, W2_r):
    n, d = x.shape
    e = edge_index.shape[1]
    ei = edge_index.astype(jnp.int32)
    src, dst, zf = _tc_split(ei, d)
    z1 = jnp.zeros((n,), jnp.float32)

    t1 = _tc_self(x, W1_r, b1)
    sc1 = _sc_aggregate(n, d, e, with_counts=True)
    parts1, cnts = sc1(x, src, dst, zf, z1)

    h = _tc_combine(parts1, cnts, t1, W1_l, relu=True)

    t2 = _tc_self(h, W2_r, b2)
    sc2 = _sc_aggregate(n, d, e, with_counts=False)
    (parts2,) = sc2(h, src, dst, zf, z1)

    out = _tc_combine(parts2, cnts, t2, W2_l, relu=False)
    return out


# speedup vs baseline: 1.0120x; 1.0120x over previous
import jax
import jax.numpy as jnp
from jax import lax
from jax.experimental import pallas as pl
from jax.experimental.pallas import tpu as pltpu
from jax.experimental.pallas import tpu_sc as plsc

NC = 2
NS = 16
L = 16
CW = 8

CH = 80
NBUF = 4
PF = 2
SD = 3

_ZR = 1000


def _sc_aggregate(n_nodes, d, n_edges, with_counts):
    nw = NC * NS
    ept = n_edges // nw
    assert ept * nw == n_edges and ept % CH == 0
    n_chunks = ept // CH
    assert PF < SD < NBUF
    assert n_nodes % _ZR == 0

    mesh = plsc.VectorSubcoreMesh(core_axis_name="c", subcore_axis_name="s")

    out_type = [jax.ShapeDtypeStruct((NC, n_nodes, d), jnp.float32)]
    if with_counts:
        out_type.append(jax.ShapeDtypeStruct((NC, n_nodes), jnp.float32))

    scratch = [
        pltpu.VMEM_SHARED((n_nodes, d), jnp.float32),
    ]
    scratch += [pltpu.SemaphoreType.DMA for _ in range(3 * NBUF)]
    scratch += [pltpu.VMEM((CH, d), jnp.float32) for _ in range(NBUF)]
    scratch += [pltpu.VMEM((CH,), jnp.int32) for _ in range(NBUF)]
    scratch += [pltpu.VMEM((CH,), jnp.int32) for _ in range(NBUF)]
    if with_counts:
        scratch += [
            pltpu.VMEM_SHARED((n_nodes,), jnp.float32),
            pltpu.VMEM((CH,), jnp.float32),
        ]

    def body(feat, src, dst, zf, z1, *rest):
        k = 1 + (1 if with_counts else 0)
        parts = rest[0]
        cnts = rest[1] if with_counts else None
        agg_sh = rest[k]
        p = k + 1
        gsem = list(rest[p:p + NBUF])
        dsem = list(rest[p + NBUF:p + 2 * NBUF])
        ssem = list(rest[p + 2 * NBUF:p + 3 * NBUF])
        p += 3 * NBUF
        rows = list(rest[p:p + NBUF])
        dring = list(rest[p + NBUF:p + 2 * NBUF])
        sring = list(rest[p + 2 * NBUF:p + 3 * NBUF])
        if with_counts:
            cnt_sh, ones_v = rest[p + 3 * NBUF:]
        else:
            cnt_sh = ones_v = None
        cid = lax.axis_index("c")
        sid = lax.axis_index("s")
        wid = cid * NS + sid
        base = wid * ept

        def issue_idx(c, slot):
            pltpu.async_copy(src.at[pl.ds(base + c * CH, CH)], sring[slot],
                             ssem[slot])
            pltpu.async_copy(dst.at[pl.ds(base + c * CH, CH)], dring[slot],
                             dsem[slot])

        def issue_gather(c, slot):
            pltpu.make_async_copy(src.at[pl.ds(base + c * CH, CH)],
                                  sring[slot], ssem[slot]).wait()
            pltpu.async_copy(feat.at[sring[slot]], rows[slot],
                             gsem[slot])

        for c0 in range(SD):
            issue_idx(c0, c0)
        for c0 in range(PF):
            issue_gather(c0, c0)

        r_lo = n_nodes // NS // 8 * 8
        r_hi = n_nodes - r_lo * (NS - 1)
        row0 = sid * r_lo

        @pl.when(sid < NS - 1)
        def _():
            pltpu.sync_copy(zf.at[pl.ds(0, r_lo)],
                            agg_sh.at[pl.ds(row0, r_lo)])

        @pl.when(sid == NS - 1)
        def _():
            pltpu.sync_copy(zf.at[pl.ds(0, r_hi)],
                            agg_sh.at[pl.ds(row0, r_hi)])
        if with_counts:
            @pl.when(sid == 0)
            def _():
                pltpu.sync_copy(z1, cnt_sh)
        if with_counts:
            for q in range(CH // L):
                ones_v[pl.ds(q * L, L)] = jnp.ones((L,), jnp.float32)
        plsc.subcore_barrier()

        def turn(c, j):
            ij = (j + SD) % NBUF
            gj = (j + PF) % NBUF

            @pl.when(c + SD < n_chunks)
            def _():
                issue_idx(c + SD, ij)

            @pl.when(c + PF < n_chunks)
            def _():
                issue_gather(c + PF, gj)
            pltpu.make_async_copy(feat.at[sring[j]], rows[j],
                                  gsem[j]).wait()
            pltpu.make_async_copy(dst.at[pl.ds(base + c * CH, CH)], dring[j],
                                  dsem[j]).wait()
            pltpu.sync_copy(rows[j], agg_sh.at[dring[j]], add=True)
            if with_counts:
                pltpu.sync_copy(ones_v, cnt_sh.at[dring[j]], add=True)

        def super_turn(t, _):
            for j in range(NBUF):
                turn(t * NBUF + j, j)
            return 0

        n_full = n_chunks // NBUF
        lax.fori_loop(0, n_full, super_turn, 0)
        for jj in range(n_chunks % NBUF):
            turn(n_full * NBUF + jj, jj)

        plsc.subcore_barrier()

        @pl.when(sid < NS - 1)
        def _():
            pltpu.sync_copy(agg_sh.at[pl.ds(row0, r_lo)],
                            parts.at[cid, pl.ds(row0, r_lo)])

        @pl.when(sid == NS - 1)
        def _():
            pltpu.sync_copy(agg_sh.at[pl.ds(row0, r_hi)],
                            parts.at[cid, pl.ds(row0, r_hi)])
        if with_counts:
            @pl.when(sid == 0)
            def _():
                pltpu.sync_copy(cnt_sh, cnts.at[cid])

    kern = pl.kernel(body, out_type=tuple(out_type), mesh=mesh,
                     scratch_types=scratch)
    return kern


_BLK = 1000


def _tc_split(ei, d):
    e = ei.shape[1]

    def body(ei_r, s_r, d_r, zf_r):
        s_r[...] = ei_r[0]
        d_r[...] = ei_r[1]
        zf_r[...] = jnp.zeros_like(zf_r)

    return pl.pallas_call(
        body,
        out_shape=[
            jax.ShapeDtypeStruct((e,), jnp.int32),
            jax.ShapeDtypeStruct((e,), jnp.int32),
            jax.ShapeDtypeStruct((_ZR, d), jnp.float32),
        ],
    )(ei)


def _tc_self(x_self, W_r, b):
    n, d = x_self.shape
    h = W_r.shape[1]

    def body(x_r, wr_r, b_r, o_r):
        o_r[...] = (jnp.dot(x_r[...], wr_r[...],
                            preferred_element_type=jnp.float32) + b_r[...])

    grid = (n // _BLK,)
    row_spec = pl.BlockSpec((_BLK, d), lambda i: (i, 0))
    w_spec = pl.BlockSpec((d, h), lambda i: (0, 0))
    b_spec = pl.BlockSpec((1, h), lambda i: (0, 0))
    return pl.pallas_call(
        body,
        grid=grid,
        in_specs=[row_spec, w_spec, b_spec],
        out_specs=pl.BlockSpec((_BLK, h), lambda i: (i, 0)),
        out_shape=jax.ShapeDtypeStruct((n, h), jnp.float32),
    )(x_self, W_r, b.reshape(1, h))


def _tc_combine(parts, cnts, t_self, W_l, relu):
    n, h = t_self.shape
    d = parts.shape[2]
    blk = _BLK
    assert n % blk == 0

    def body(p_r, c_r, t_r, wl_r, o_r):
        agg = p_r[0] + p_r[1]
        cnt = c_r[0] + c_r[1]
        recip = (1.0 / jnp.maximum(cnt, 1.0))[:, None]
        mean = agg * recip
        out = (jnp.dot(mean, wl_r[...], preferred_element_type=jnp.float32)
               + t_r[...])
        if relu:
            out = jnp.maximum(out, 0.0)
        o_r[...] = out

    return pl.pallas_call(
        body,
        out_shape=jax.ShapeDtypeStruct((n, h), jnp.float32),
    )(parts, cnts, t_self, W_l)


def kernel(x, edge_index, W1_l, b1, W1_r, W2_l, b2, W2_r):
    n, d = x.shape
    e = edge_index.shape[1]
    ei = edge_index.astype(jnp.int32)
    src, dst, zf = _tc_split(ei, d)
    z1 = jnp.zeros((n,), jnp.float32)

    t1 = _tc_self(x, W1_r, b1)
    sc1 = _sc_aggregate(n, d, e, with_counts=True)
    parts1, cnts = sc1(x, src, dst, zf, z1)

    h = _tc_combine(parts1, cnts, t1, W1_l, relu=True)

    t2 = _tc_self(h, W2_r, b2)
    sc2 = _sc_aggregate(n, d, e, with_counts=False)
    (parts2,) = sc2(h, src, dst, zf, z1)

    out = _tc_combine(parts2, cnts, t2, W2_l, relu=False)
    return out
